# Initial kernel scaffold; baseline (speedup 1.0000x reference)
import jax
import jax.numpy as jnp
from jax.experimental import pallas as pl


def kernel(x, categorical_covariates, edge_index, edge_weight, emb0, emb1, W_map, b_map, Wr0, br0, Wroot0, Wr1, br1, Wroot1, Wr2, br2, Wroot2):
    raise NotImplementedError("write your pallas kernel here")


# speedup vs baseline: 45.0285x; 45.0285x over previous
import functools
import jax
import jax.numpy as jnp
from jax import lax
from jax.experimental import pallas as pl
from jax.experimental.pallas import tpu as pltpu
from jax.experimental.pallas import tpu_sc as plsc

N = 10000
E = 160000
B = 8

NC = 2
NS = 16
NW = NC * NS
K = 128
E_PAD = 163840
CHUNKS = E_PAD // (NW * K)
NT = 10240
_MESH = dict(core_axis_name="c", subcore_axis_name="s",
             num_cores=NC, num_subcores=NS)
_CP = pltpu.CompilerParams(needs_layout_passes=False)


@functools.lru_cache(maxsize=None)
def _make_spmv_narrow():
  TAB = 8 * NT
  ACC = 9 * NT
  SEG_T = TAB // NS
  SEG_A = ACC // NS

  @functools.partial(
      pl.kernel,
      mesh=plsc.VectorSubcoreMesh(**_MESH),
      compiler_params=_CP,
      out_type=jax.ShapeDtypeStruct((NC, ACC), jnp.float32),
      scratch_types=[
          pltpu.VMEM((K,), jnp.int32),
          pltpu.VMEM((K,), jnp.int32),
          pltpu.VMEM((K,), jnp.float32),
          pltpu.VMEM((K,), jnp.int32),
          pltpu.VMEM((K,), jnp.int32),
          pltpu.VMEM((K,), jnp.float32),
          pltpu.VMEM((K,), jnp.float32),
          pltpu.VMEM_SHARED((TAB,), jnp.float32),
          pltpu.VMEM_SHARED((ACC,), jnp.float32),
          pltpu.SemaphoreType.DMA,
      ],
  )
  def spmv(x_hbm, src_hbm, dst_hbm, ew_hbm, out_hbm,
           src_v, dst_v, ew_v, gidx, sidx, vals_v, aux_v, tab, acc, sem):
    cid = lax.axis_index("c")
    sid = lax.axis_index("s")
    wid = sid * NC + cid

    for g in range(K // 16):
      aux_v[pl.ds(g * 16, 16)] = jnp.zeros((16,), jnp.float32)
    def zacc(j, carry):
      pltpu.sync_copy(aux_v, acc.at[pl.ds(sid * SEG_A + j * K, K)])
      return carry
    lax.fori_loop(0, SEG_A // K, zacc, 0)
    pltpu.sync_copy(x_hbm.at[pl.ds(sid * SEG_T, SEG_T)],
                    tab.at[pl.ds(sid * SEG_T, SEG_T)])
    plsc.subcore_barrier()

    for g in range(K // 16):
      aux_v[pl.ds(g * 16, 16)] = jnp.ones((16,), jnp.float32)

    def chunk(ch, carry):
      pltpu.sync_copy(src_hbm.at[wid, ch], src_v)
      pltpu.sync_copy(dst_hbm.at[wid, ch], dst_v)
      pltpu.sync_copy(ew_hbm.at[wid, ch], ew_v)
      for c in range(8):
        for g in range(K // 16):
          s = pl.ds(g * 16, 16)
          gidx[s] = src_v[s] + (c * NT)
        pltpu.async_copy(tab.at[gidx], vals_v, sem).wait()
        for g in range(K // 16):
          s = pl.ds(g * 16, 16)
          vals_v[s] = vals_v[s] * ew_v[s]
          sidx[s] = dst_v[s] + (c * NT)
        pltpu.sync_copy(vals_v, acc.at[sidx], add=True)
      for g in range(K // 16):
        s = pl.ds(g * 16, 16)
        sidx[s] = dst_v[s] + (8 * NT)
      pltpu.sync_copy(aux_v, acc.at[sidx], add=True)
      return carry
    lax.fori_loop(0, CHUNKS, chunk, 0)
    plsc.subcore_barrier()

    pltpu.sync_copy(acc.at[pl.ds(sid * SEG_A, SEG_A)],
                    out_hbm.at[cid, pl.ds(sid * SEG_A, SEG_A)])

  return spmv


@functools.lru_cache(maxsize=None)
def _make_spmv_wide():
  RPT = NT // NS

  @functools.partial(
      pl.kernel,
      mesh=plsc.VectorSubcoreMesh(**_MESH),
      compiler_params=_CP,
      out_type=jax.ShapeDtypeStruct((NC, NT, 128), jnp.float32),
      scratch_types=[
          pltpu.VMEM((K,), jnp.int32),
          pltpu.VMEM((K,), jnp.int32),
          pltpu.VMEM((K,), jnp.float32),
          pltpu.VMEM((K, 128), jnp.float32),
          pltpu.VMEM_SHARED((NT, 128), jnp.float32),
          pltpu.SemaphoreType.DMA,
      ],
  )
  def spmv(x_hbm, src_hbm, dst_hbm, ew_hbm, out_hbm,
           src_v, dst_v, ew_v, rows_v, acc, sem):
    cid = lax.axis_index("c")
    sid = lax.axis_index("s")
    wid = sid * NC + cid

    def zrow(i, carry):
      for g in range(8):
        rows_v[i, pl.ds(g * 16, 16)] = jnp.zeros((16,), jnp.float32)
      return carry
    lax.fori_loop(0, K, zrow, 0)
    def zacc(j, carry):
      pltpu.sync_copy(rows_v, acc.at[pl.ds(sid * RPT + j * K, K)])
      return carry
    lax.fori_loop(0, RPT // K, zacc, 0)
    plsc.subcore_barrier()

    def chunk(ch, carry):
      pltpu.sync_copy(src_hbm.at[wid, ch], src_v)
      pltpu.sync_copy(dst_hbm.at[wid, ch], dst_v)
      pltpu.sync_copy(ew_hbm.at[wid, ch], ew_v)
      pltpu.async_copy(x_hbm.at[src_v], rows_v, sem).wait()
      for g in range(K // 16):
        ew16 = ew_v[pl.ds(g * 16, 16)]
        for j in range(16):
          kk = g * 16 + j
          ewk = jnp.full((16,), ew16[j], jnp.float32)
          for gg in range(8):
            s = pl.ds(gg * 16, 16)
            rows_v[kk, s] = rows_v[kk, s] * ewk
      pltpu.sync_copy(rows_v, acc.at[dst_v], add=True)
      return carry
    lax.fori_loop(0, CHUNKS, chunk, 0)
    plsc.subcore_barrier()

    pltpu.sync_copy(acc.at[pl.ds(sid * RPT, RPT)],
                    out_hbm.at[cid, pl.ds(sid * RPT, RPT)])

  return spmv


_BLK = 1024
_GRID = NT // _BLK


def _stageA_body(x_ref, cc_ref, e0_ref, e1_ref, W_ref, bm_ref, out_ref):
  cc0 = cc_ref[:, 0:1]
  cc1 = cc_ref[:, 1:2]
  oh0 = (lax.broadcasted_iota(jnp.int32, (8, 10), 1) == cc0).astype(jnp.float32)
  oh1 = (lax.broadcasted_iota(jnp.int32, (8, 4), 1) == cc1).astype(jnp.float32)
  z = jnp.concatenate([
      x_ref[...],
      jnp.dot(oh0, e0_ref[...], preferred_element_type=jnp.float32),
      jnp.dot(oh1, e1_ref[...], preferred_element_type=jnp.float32),
  ], axis=1)
  out_ref[...] = jnp.dot(z, W_ref[...],
                         preferred_element_type=jnp.float32) + bm_ref[...]


def _stageB_body(s0_ref, mT_ref, Wr0_ref, br0_ref, Wroot0_ref, Wr1_ref,
                 Wroot1_ref, p1_ref, r1_ref, inv_ref):
  s = s0_ref[0] + s0_ref[1]
  cnt = s[:, 8:9]
  inv = 1.0 / jnp.maximum(cnt, 1.0)
  a0 = s[:, 0:8] * inv
  mt = mT_ref[...]
  for b in range(B):
    u = (a0[:, b:b + 1] * Wr0_ref[...] + mt[:, b:b + 1] * Wroot0_ref[...]
         + br0_ref[...])
    h1 = jnp.maximum(u, 0.0)
    p1_ref[:, b * 16:(b + 1) * 16] = jnp.dot(
        h1, Wr1_ref[...], preferred_element_type=jnp.float32)
    r1_ref[:, b * 16:(b + 1) * 16] = jnp.dot(
        h1, Wroot1_ref[...], preferred_element_type=jnp.float32)
  inv_ref[...] = jnp.broadcast_to(inv, (_BLK, 8))


def _stageC_body(s1_ref, r1_ref, inv_ref, br1t_ref, wr2t_ref, wroot2t_ref,
                 p2_ref, r2_ref):
  inv = inv_ref[:, 0:1]
  agg1 = (s1_ref[0] + s1_ref[1]) * inv
  h2 = jnp.maximum(agg1 + r1_ref[...] + br1t_ref[...], 0.0)
  p2cols = []
  r2cols = []
  for b in range(B):
    hb = h2[:, b * 16:(b + 1) * 16]
    p2cols.append(jnp.sum(hb * wr2t_ref[...], axis=1, keepdims=True))
    r2cols.append(jnp.sum(hb * wroot2t_ref[...], axis=1, keepdims=True))
  p2_ref[...] = jnp.concatenate(p2cols, axis=1)
  r2_ref[...] = jnp.concatenate(r2cols, axis=1)


def _stageD_body(s2_ref, inv_ref, r2_ref, br2_ref, out_ref):
  agg2 = (s2_ref[0] + s2_ref[1]) * inv_ref[:, 0:1]
  out_ref[...] = agg2 + r2_ref[...] + br2_ref[0, 0]


def _nblk(i):
  return (i, 0)


def kernel(x, categorical_covariates, edge_index, edge_weight, emb0, emb1,
           W_map, b_map, Wr0, br0, Wroot0, Wr1, br1, Wroot1, Wr2, br2,
           Wroot2):
  f32 = jnp.float32
  cc = categorical_covariates.astype(jnp.int32)
  Wp = jnp.pad(W_map, ((0, 0), (0, NT - N)))
  bm = jnp.pad(b_map, (0, NT - N)).reshape(1, NT)
  src = edge_index[0].astype(jnp.int32)
  dst = edge_index[1].astype(jnp.int32)
  ew = edge_weight.astype(f32)
  src_p = jnp.pad(src, (0, E_PAD - E)).reshape(NW, CHUNKS, K)
  dst_p = jnp.pad(dst, (0, E_PAD - E),
                  constant_values=N).reshape(NW, CHUNKS, K)
  ew_p = jnp.pad(ew, (0, E_PAD - E)).reshape(NW, CHUNKS, K)
  br1t = jnp.tile(br1, B).reshape(1, 128)
  wr2t = Wr2[:, 0].reshape(1, 16)
  wroot2t = Wroot2[:, 0].reshape(1, 16)
  br2_2d = br2.reshape(1, 1)

  full = lambda s: pl.BlockSpec(s, lambda i: tuple(0 for _ in s))

  m = pl.pallas_call(
      _stageA_body,
      grid=(_GRID,),
      in_specs=[full((8, 128)), full((8, 2)), full((10, 4)), full((4, 3)),
                pl.BlockSpec((135, _BLK), lambda i: (0, i)),
                pl.BlockSpec((1, _BLK), lambda i: (0, i))],
      out_specs=pl.BlockSpec((8, _BLK), lambda i: (0, i)),
      out_shape=jax.ShapeDtypeStruct((8, NT), f32),
  )(x, cc, emb0, emb1, Wp, bm)

  s0 = _make_spmv_narrow()(m.reshape(8 * NT), src_p, dst_p, ew_p)
  s0n = s0.reshape(2, 9, NT).transpose(0, 2, 1)

  p1, r1, invb = pl.pallas_call(
      _stageB_body,
      grid=(_GRID,),
      in_specs=[pl.BlockSpec((2, _BLK, 9), lambda i: (0, i, 0)),
                pl.BlockSpec((_BLK, 8), _nblk),
                full((1, 32)), full((1, 32)), full((1, 32)),
                full((32, 16)), full((32, 16))],
      out_specs=[pl.BlockSpec((_BLK, 128), _nblk),
                 pl.BlockSpec((_BLK, 128), _nblk),
                 pl.BlockSpec((_BLK, 8), _nblk)],
      out_shape=[jax.ShapeDtypeStruct((NT, 128), f32),
                 jax.ShapeDtypeStruct((NT, 128), f32),
                 jax.ShapeDtypeStruct((NT, 8), f32)],
  )(s0n, m.T, Wr0, br0.reshape(1, 32), Wroot0, Wr1, Wroot1)

  s1 = _make_spmv_wide()(p1, src_p, dst_p, ew_p)

  p2, r2 = pl.pallas_call(
      _stageC_body,
      grid=(_GRID,),
      in_specs=[pl.BlockSpec((2, _BLK, 128), lambda i: (0, i, 0)),
                pl.BlockSpec((_BLK, 128), _nblk),
                pl.BlockSpec((_BLK, 8), _nblk),
                full((1, 128)), full((1, 16)), full((1, 16))],
      out_specs=[pl.BlockSpec((_BLK, 8), _nblk),
                 pl.BlockSpec((_BLK, 8), _nblk)],
      out_shape=[jax.ShapeDtypeStruct((NT, 8), f32),
                 jax.ShapeDtypeStruct((NT, 8), f32)],
  )(s1, r1, invb, br1t, wr2t, wroot2t)

  s2 = _make_spmv_narrow()(p2.T.reshape(8 * NT), src_p, dst_p, ew_p)
  s2n = s2.reshape(2, 9, NT)[:, :8].transpose(0, 2, 1)

  outT = pl.pallas_call(
      _stageD_body,
      grid=(_GRID,),
      in_specs=[pl.BlockSpec((2, _BLK, 8), lambda i: (0, i, 0)),
                pl.BlockSpec((_BLK, 8), _nblk),
                pl.BlockSpec((_BLK, 8), _nblk),
                full((1, 1))],
      out_specs=pl.BlockSpec((_BLK, 8), _nblk),
      out_shape=jax.ShapeDtypeStruct((NT, 8), f32),
  )(s2n, invb, r2, br2_2d)

  return outT[:N].T


# speedup vs baseline: 52.4549x; 1.1649x over previous
import functools
import jax
import jax.numpy as jnp
from jax import lax
from jax.experimental import pallas as pl
from jax.experimental.pallas import tpu as pltpu
from jax.experimental.pallas import tpu_sc as plsc

N = 10000
E = 160000
B = 8

NC = 2
NS = 16
NW = NC * NS
K = 128
E_PAD = 163840
CHUNKS = E_PAD // (NW * K)
NT = 10240
_MESH = dict(core_axis_name="c", subcore_axis_name="s",
             num_cores=NC, num_subcores=NS)
_CP = pltpu.CompilerParams(needs_layout_passes=False)


@functools.lru_cache(maxsize=None)
def _make_spmv_narrow():
  TAB = 8 * NT
  ACC = 9 * NT
  SEG_T = TAB // NS
  SEG_A = ACC // NS

  @functools.partial(
      pl.kernel,
      mesh=plsc.VectorSubcoreMesh(**_MESH),
      compiler_params=_CP,
      out_type=jax.ShapeDtypeStruct((NC, ACC), jnp.float32),
      scratch_types=(
          [pltpu.VMEM((K,), jnp.int32),
           pltpu.VMEM((K,), jnp.int32),
           pltpu.VMEM((K,), jnp.float32)]
          + [pltpu.VMEM((K,), jnp.int32) for _ in range(8)]
          + [pltpu.VMEM((K,), jnp.int32) for _ in range(9)]
          + [pltpu.VMEM((K,), jnp.float32) for _ in range(8)]
          + [pltpu.VMEM((K,), jnp.float32),
             pltpu.VMEM_SHARED((TAB,), jnp.float32),
             pltpu.VMEM_SHARED((ACC,), jnp.float32),
             pltpu.SemaphoreType.DMA,
             pltpu.SemaphoreType.DMA]
      ),
  )
  def spmv(x_hbm, src_hbm, dst_hbm, ew_hbm, out_hbm, *scr):
    src_v, dst_v, ew_v = scr[0:3]
    gidx = scr[3:11]
    sidx = scr[11:20]
    vals = scr[20:28]
    aux_v, tab, acc, gsem, ssem = scr[28:33]
    cid = lax.axis_index("c")
    sid = lax.axis_index("s")
    wid = sid * NC + cid

    for g in range(K // 16):
      aux_v[pl.ds(g * 16, 16)] = jnp.zeros((16,), jnp.float32)
    def zacc(j, carry):
      pltpu.sync_copy(aux_v, acc.at[pl.ds(sid * SEG_A + j * K, K)])
      return carry
    lax.fori_loop(0, SEG_A // K, zacc, 0)
    pltpu.sync_copy(x_hbm.at[pl.ds(sid * SEG_T, SEG_T)],
                    tab.at[pl.ds(sid * SEG_T, SEG_T)])
    plsc.subcore_barrier()

    for g in range(K // 16):
      aux_v[pl.ds(g * 16, 16)] = jnp.ones((16,), jnp.float32)

    def chunk(ch, carry):
      pltpu.sync_copy(src_hbm.at[wid, ch], src_v)
      pltpu.sync_copy(dst_hbm.at[wid, ch], dst_v)
      pltpu.sync_copy(ew_hbm.at[wid, ch], ew_v)
      gh = []
      for c in range(8):
        for g in range(K // 16):
          s = pl.ds(g * 16, 16)
          gidx[c][s] = src_v[s] + (c * NT)
        gh.append(pltpu.async_copy(tab.at[gidx[c]], vals[c], gsem))
      for c in range(9):
        for g in range(K // 16):
          s = pl.ds(g * 16, 16)
          sidx[c][s] = dst_v[s] + (c * NT)
      sh = []
      for c in range(8):
        gh[c].wait()
        for g in range(K // 16):
          s = pl.ds(g * 16, 16)
          vals[c][s] = vals[c][s] * ew_v[s]
        sh.append(pltpu.async_copy(vals[c], acc.at[sidx[c]], ssem, add=True))
      sh.append(pltpu.async_copy(aux_v, acc.at[sidx[8]], ssem, add=True))
      for h in sh:
        h.wait()
      return carry
    lax.fori_loop(0, CHUNKS, chunk, 0)
    plsc.subcore_barrier()

    pltpu.sync_copy(acc.at[pl.ds(sid * SEG_A, SEG_A)],
                    out_hbm.at[cid, pl.ds(sid * SEG_A, SEG_A)])

  return spmv


@functools.lru_cache(maxsize=None)
def _make_spmv_wide():
  RPT = NT // NS

  @functools.partial(
      pl.kernel,
      mesh=plsc.VectorSubcoreMesh(**_MESH),
      compiler_params=_CP,
      out_type=jax.ShapeDtypeStruct((NC, NT, 128), jnp.float32),
      scratch_types=[
          pltpu.VMEM((K,), jnp.int32),
          pltpu.VMEM((K,), jnp.int32),
          pltpu.VMEM((K,), jnp.float32),
          pltpu.VMEM((K,), jnp.int32),
          pltpu.VMEM((K,), jnp.int32),
          pltpu.VMEM((K,), jnp.float32),
          pltpu.VMEM((K, 128), jnp.float32),
          pltpu.VMEM((K, 128), jnp.float32),
          pltpu.VMEM_SHARED((NT, 128), jnp.float32),
          pltpu.SemaphoreType.DMA,
          pltpu.SemaphoreType.DMA,
      ],
  )
  def spmv(x_hbm, src_hbm, dst_hbm, ew_hbm, out_hbm,
           src0, dst0, ew0, src1, dst1, ew1, rows0, rows1, acc, gsem, ssem):
    cid = lax.axis_index("c")
    sid = lax.axis_index("s")
    wid = sid * NC + cid
    srcs = (src0, src1)
    dsts = (dst0, dst1)
    ews = (ew0, ew1)
    rows = (rows0, rows1)

    def zrow(i, carry):
      for g in range(8):
        rows0[i, pl.ds(g * 16, 16)] = jnp.zeros((16,), jnp.float32)
      return carry
    lax.fori_loop(0, K, zrow, 0)
    def zacc(j, carry):
      pltpu.sync_copy(rows0, acc.at[pl.ds(sid * RPT + j * K, K)])
      return carry
    lax.fori_loop(0, RPT // K, zacc, 0)
    plsc.subcore_barrier()

    def mul(b):
      for g in range(K // 16):
        ew16 = ews[b][pl.ds(g * 16, 16)]
        for j in range(16):
          kk = g * 16 + j
          ewk = jnp.full((16,), ew16[j], jnp.float32)
          for gg in range(8):
            s = pl.ds(gg * 16, 16)
            rows[b][kk, s] = rows[b][kk, s] * ewk

    def pair(i, carry):
      gh = []
      for b in range(2):
        ch = i * 2 + b
        pltpu.sync_copy(src_hbm.at[wid, ch], srcs[b])
        pltpu.sync_copy(dst_hbm.at[wid, ch], dsts[b])
        pltpu.sync_copy(ew_hbm.at[wid, ch], ews[b])
        gh.append(pltpu.async_copy(x_hbm.at[srcs[b]], rows[b], gsem))
      sh = []
      for b in range(2):
        gh[b].wait()
        mul(b)
        sh.append(pltpu.async_copy(rows[b], acc.at[dsts[b]], ssem, add=True))
      for h in sh:
        h.wait()
      return carry
    lax.fori_loop(0, CHUNKS // 2, pair, 0)
    plsc.subcore_barrier()

    pltpu.sync_copy(acc.at[pl.ds(sid * RPT, RPT)],
                    out_hbm.at[cid, pl.ds(sid * RPT, RPT)])

  return spmv


_BLK = 1024
_GRID = NT // _BLK


def _stageA_body(x_ref, cc_ref, e0_ref, e1_ref, W_ref, bm_ref, out_ref):
  cc0 = cc_ref[:, 0:1]
  cc1 = cc_ref[:, 1:2]
  oh0 = (lax.broadcasted_iota(jnp.int32, (8, 10), 1) == cc0).astype(jnp.float32)
  oh1 = (lax.broadcasted_iota(jnp.int32, (8, 4), 1) == cc1).astype(jnp.float32)
  z = jnp.concatenate([
      x_ref[...],
      jnp.dot(oh0, e0_ref[...], preferred_element_type=jnp.float32),
      jnp.dot(oh1, e1_ref[...], preferred_element_type=jnp.float32),
  ], axis=1)
  out_ref[...] = jnp.dot(z, W_ref[...],
                         preferred_element_type=jnp.float32) + bm_ref[...]


def _stageB_body(s0_ref, mT_ref, Wr0_ref, br0_ref, Wroot0_ref, Wr1_ref,
                 Wroot1_ref, p1_ref, r1_ref, inv_ref):
  s = s0_ref[0] + s0_ref[1]
  cnt = s[:, 8:9]
  inv = 1.0 / jnp.maximum(cnt, 1.0)
  a0 = s[:, 0:8] * inv
  mt = mT_ref[...]
  for b in range(B):
    u = (a0[:, b:b + 1] * Wr0_ref[...] + mt[:, b:b + 1] * Wroot0_ref[...]
         + br0_ref[...])
    h1 = jnp.maximum(u, 0.0)
    p1_ref[:, b * 16:(b + 1) * 16] = jnp.dot(
        h1, Wr1_ref[...], preferred_element_type=jnp.float32)
    r1_ref[:, b * 16:(b + 1) * 16] = jnp.dot(
        h1, Wroot1_ref[...], preferred_element_type=jnp.float32)
  inv_ref[...] = jnp.broadcast_to(inv, (_BLK, 8))


def _stageC_body(s1_ref, r1_ref, inv_ref, br1t_ref, wr2t_ref, wroot2t_ref,
                 p2_ref, r2_ref):
  inv = inv_ref[:, 0:1]
  agg1 = (s1_ref[0] + s1_ref[1]) * inv
  h2 = jnp.maximum(agg1 + r1_ref[...] + br1t_ref[...], 0.0)
  p2cols = []
  r2cols = []
  for b in range(B):
    hb = h2[:, b * 16:(b + 1) * 16]
    p2cols.append(jnp.sum(hb * wr2t_ref[...], axis=1, keepdims=True))
    r2cols.append(jnp.sum(hb * wroot2t_ref[...], axis=1, keepdims=True))
  p2_ref[...] = jnp.concatenate(p2cols, axis=1)
  r2_ref[...] = jnp.concatenate(r2cols, axis=1)


def _stageD_body(s2_ref, inv_ref, r2_ref, br2_ref, out_ref):
  agg2 = (s2_ref[0] + s2_ref[1]) * inv_ref[:, 0:1]
  out_ref[...] = agg2 + r2_ref[...] + br2_ref[0, 0]


def _nblk(i):
  return (i, 0)


def kernel(x, categorical_covariates, edge_index, edge_weight, emb0, emb1,
           W_map, b_map, Wr0, br0, Wroot0, Wr1, br1, Wroot1, Wr2, br2,
           Wroot2):
  f32 = jnp.float32
  cc = categorical_covariates.astype(jnp.int32)
  Wp = jnp.pad(W_map, ((0, 0), (0, NT - N)))
  bm = jnp.pad(b_map, (0, NT - N)).reshape(1, NT)
  src = edge_index[0].astype(jnp.int32)
  dst = edge_index[1].astype(jnp.int32)
  ew = edge_weight.astype(f32)
  src_p = jnp.pad(src, (0, E_PAD - E)).reshape(NW, CHUNKS, K)
  dst_p = jnp.pad(dst, (0, E_PAD - E),
                  constant_values=N).reshape(NW, CHUNKS, K)
  ew_p = jnp.pad(ew, (0, E_PAD - E)).reshape(NW, CHUNKS, K)
  br1t = jnp.tile(br1, B).reshape(1, 128)
  wr2t = Wr2[:, 0].reshape(1, 16)
  wroot2t = Wroot2[:, 0].reshape(1, 16)
  br2_2d = br2.reshape(1, 1)

  full = lambda s: pl.BlockSpec(s, lambda i: tuple(0 for _ in s))

  m = pl.pallas_call(
      _stageA_body,
      grid=(_GRID,),
      in_specs=[full((8, 128)), full((8, 2)), full((10, 4)), full((4, 3)),
                pl.BlockSpec((135, _BLK), lambda i: (0, i)),
                pl.BlockSpec((1, _BLK), lambda i: (0, i))],
      out_specs=pl.BlockSpec((8, _BLK), lambda i: (0, i)),
      out_shape=jax.ShapeDtypeStruct((8, NT), f32),
  )(x, cc, emb0, emb1, Wp, bm)

  s0 = _make_spmv_narrow()(m.reshape(8 * NT), src_p, dst_p, ew_p)
  s0n = s0.reshape(2, 9, NT).transpose(0, 2, 1)

  p1, r1, invb = pl.pallas_call(
      _stageB_body,
      grid=(_GRID,),
      in_specs=[pl.BlockSpec((2, _BLK, 9), lambda i: (0, i, 0)),
                pl.BlockSpec((_BLK, 8), _nblk),
                full((1, 32)), full((1, 32)), full((1, 32)),
                full((32, 16)), full((32, 16))],
      out_specs=[pl.BlockSpec((_BLK, 128), _nblk),
                 pl.BlockSpec((_BLK, 128), _nblk),
                 pl.BlockSpec((_BLK, 8), _nblk)],
      out_shape=[jax.ShapeDtypeStruct((NT, 128), f32),
                 jax.ShapeDtypeStruct((NT, 128), f32),
                 jax.ShapeDtypeStruct((NT, 8), f32)],
  )(s0n, m.T, Wr0, br0.reshape(1, 32), Wroot0, Wr1, Wroot1)

  s1 = _make_spmv_wide()(p1, src_p, dst_p, ew_p)

  p2, r2 = pl.pallas_call(
      _stageC_body,
      grid=(_GRID,),
      in_specs=[pl.BlockSpec((2, _BLK, 128), lambda i: (0, i, 0)),
                pl.BlockSpec((_BLK, 128), _nblk),
                pl.BlockSpec((_BLK, 8), _nblk),
                full((1, 128)), full((1, 16)), full((1, 16))],
      out_specs=[pl.BlockSpec((_BLK, 8), _nblk),
                 pl.BlockSpec((_BLK, 8), _nblk)],
      out_shape=[jax.ShapeDtypeStruct((NT, 8), f32),
                 jax.ShapeDtypeStruct((NT, 8), f32)],
  )(s1, r1, invb, br1t, wr2t, wroot2t)

  s2 = _make_spmv_narrow()(p2.T.reshape(8 * NT), src_p, dst_p, ew_p)
  s2n = s2.reshape(2, 9, NT)[:, :8].transpose(0, 2, 1)

  outT = pl.pallas_call(
      _stageD_body,
      grid=(_GRID,),
      in_specs=[pl.BlockSpec((2, _BLK, 8), lambda i: (0, i, 0)),
                pl.BlockSpec((_BLK, 8), _nblk),
                pl.BlockSpec((_BLK, 8), _nblk),
                full((1, 1))],
      out_specs=pl.BlockSpec((_BLK, 8), _nblk),
      out_shape=jax.ShapeDtypeStruct((NT, 8), f32),
  )(s2n, invb, r2, br2_2d)

  return outT[:N].T


# speedup vs baseline: 59.7425x; 1.1389x over previous
import functools
import jax
import jax.numpy as jnp
from jax import lax
from jax.experimental import pallas as pl
from jax.experimental.pallas import tpu as pltpu
from jax.experimental.pallas import tpu_sc as plsc

N = 10000
E = 160000
B = 8

NC = 2
NS = 16
NW = NC * NS
K = 128
E_PAD = 163840
CHUNKS = E_PAD // (NW * K)
KW = 80
E_PADW = 161280
CHUNKSW = E_PADW // (NW * KW)
NT = 10240
_MESH = dict(core_axis_name="c", subcore_axis_name="s",
             num_cores=NC, num_subcores=NS)
_CP = pltpu.CompilerParams(needs_layout_passes=False)


@functools.lru_cache(maxsize=None)
def _make_spmv_narrow():
  TAB = 8 * NT
  ACC = 9 * NT
  SEG_T = TAB // NS
  SEG_A = ACC // NS

  @functools.partial(
      pl.kernel,
      mesh=plsc.VectorSubcoreMesh(**_MESH),
      compiler_params=_CP,
      out_type=jax.ShapeDtypeStruct((NC, ACC), jnp.float32),
      scratch_types=(
          [pltpu.VMEM((K,), jnp.int32),
           pltpu.VMEM((K,), jnp.int32),
           pltpu.VMEM((K,), jnp.float32)]
          + [pltpu.VMEM((K,), jnp.int32) for _ in range(8)]
          + [pltpu.VMEM((K,), jnp.int32) for _ in range(9)]
          + [pltpu.VMEM((K,), jnp.float32) for _ in range(8)]
          + [pltpu.VMEM((K,), jnp.float32),
             pltpu.VMEM_SHARED((TAB,), jnp.float32),
             pltpu.VMEM_SHARED((ACC,), jnp.float32),
             pltpu.SemaphoreType.DMA,
             pltpu.SemaphoreType.DMA]
      ),
  )
  def spmv(x_hbm, src_hbm, dst_hbm, ew_hbm, out_hbm, *scr):
    src_v, dst_v, ew_v = scr[0:3]
    gidx = scr[3:11]
    sidx = scr[11:20]
    vals = scr[20:28]
    aux_v, tab, acc, gsem, ssem = scr[28:33]
    cid = lax.axis_index("c")
    sid = lax.axis_index("s")
    wid = sid * NC + cid

    for g in range(K // 16):
      aux_v[pl.ds(g * 16, 16)] = jnp.zeros((16,), jnp.float32)
    def zacc(j, carry):
      pltpu.sync_copy(aux_v, acc.at[pl.ds(sid * SEG_A + j * K, K)])
      return carry
    lax.fori_loop(0, SEG_A // K, zacc, 0)
    pltpu.sync_copy(x_hbm.at[pl.ds(sid * SEG_T, SEG_T)],
                    tab.at[pl.ds(sid * SEG_T, SEG_T)])
    plsc.subcore_barrier()

    for g in range(K // 16):
      aux_v[pl.ds(g * 16, 16)] = jnp.ones((16,), jnp.float32)

    def chunk(ch, carry):
      pltpu.sync_copy(src_hbm.at[wid, ch], src_v)
      pltpu.sync_copy(dst_hbm.at[wid, ch], dst_v)
      pltpu.sync_copy(ew_hbm.at[wid, ch], ew_v)
      gh = []
      for c in range(8):
        for g in range(K // 16):
          s = pl.ds(g * 16, 16)
          gidx[c][s] = src_v[s] + (c * NT)
        gh.append(pltpu.async_copy(tab.at[gidx[c]], vals[c], gsem))
      for c in range(9):
        for g in range(K // 16):
          s = pl.ds(g * 16, 16)
          sidx[c][s] = dst_v[s] + (c * NT)
      sh = []
      for c in range(8):
        gh[c].wait()
        for g in range(K // 16):
          s = pl.ds(g * 16, 16)
          vals[c][s] = vals[c][s] * ew_v[s]
        sh.append(pltpu.async_copy(vals[c], acc.at[sidx[c]], ssem, add=True))
      sh.append(pltpu.async_copy(aux_v, acc.at[sidx[8]], ssem, add=True))
      for h in sh:
        h.wait()
      return carry
    lax.fori_loop(0, CHUNKS, chunk, 0)
    plsc.subcore_barrier()

    pltpu.sync_copy(acc.at[pl.ds(sid * SEG_A, SEG_A)],
                    out_hbm.at[cid, pl.ds(sid * SEG_A, SEG_A)])

  return spmv


@functools.lru_cache(maxsize=None)
def _make_spmv_wide():
  RPT = NT // NS

  @functools.partial(
      pl.kernel,
      mesh=plsc.VectorSubcoreMesh(**_MESH),
      compiler_params=_CP,
      out_type=jax.ShapeDtypeStruct((NC, NT, 128), jnp.float32),
      scratch_types=(
          [pltpu.VMEM((KW,), jnp.int32) for _ in range(3)]
          + [pltpu.VMEM((KW,), jnp.int32) for _ in range(3)]
          + [pltpu.VMEM((KW,), jnp.float32) for _ in range(3)]
          + [pltpu.VMEM((KW, 128), jnp.float32) for _ in range(3)]
          + [pltpu.VMEM_SHARED((NT, 128), jnp.float32),
             pltpu.SemaphoreType.DMA,
             pltpu.SemaphoreType.DMA]
      ),
  )
  def spmv(x_hbm, src_hbm, dst_hbm, ew_hbm, out_hbm, *scr):
    srcs = scr[0:3]
    dsts = scr[3:6]
    ews = scr[6:9]
    rows = scr[9:12]
    acc, gsem, ssem = scr[12:15]
    cid = lax.axis_index("c")
    sid = lax.axis_index("s")
    wid = sid * NC + cid

    def zrow(i, carry):
      for g in range(8):
        rows[0][i, pl.ds(g * 16, 16)] = jnp.zeros((16,), jnp.float32)
      return carry
    lax.fori_loop(0, KW, zrow, 0)
    def zacc(j, carry):
      pltpu.sync_copy(rows[0], acc.at[pl.ds(sid * RPT + j * KW, KW)])
      return carry
    lax.fori_loop(0, RPT // KW, zacc, 0)
    plsc.subcore_barrier()

    def mul(b):
      def mgroup(g, carry):
        ew16 = ews[b][pl.ds(g * 16, 16)]
        for j in range(16):
          ewk = jnp.full((16,), ew16[j], jnp.float32)
          for gg in range(8):
            s = pl.ds(gg * 16, 16)
            rows[b][g * 16 + j, s] = rows[b][g * 16 + j, s] * ewk
        return carry
      lax.fori_loop(0, KW // 16, mgroup, 0)

    def quad(i, carry):
      gh = []
      for b in range(3):
        ch = i * 3 + b
        pltpu.sync_copy(src_hbm.at[wid, ch], srcs[b])
        pltpu.sync_copy(dst_hbm.at[wid, ch], dsts[b])
        pltpu.sync_copy(ew_hbm.at[wid, ch], ews[b])
        gh.append(pltpu.async_copy(x_hbm.at[srcs[b]], rows[b], gsem))
      sh = []
      for b in range(3):
        gh[b].wait()
        mul(b)
        sh.append(pltpu.async_copy(rows[b], acc.at[dsts[b]], ssem, add=True))
      for h in sh:
        h.wait()
      return carry
    lax.fori_loop(0, CHUNKSW // 3, quad, 0)
    plsc.subcore_barrier()

    pltpu.sync_copy(acc.at[pl.ds(sid * RPT, RPT)],
                    out_hbm.at[cid, pl.ds(sid * RPT, RPT)])

  return spmv


_BLK = 1024
_GRID = NT // _BLK


def _stageA_body(x_ref, cc_ref, e0_ref, e1_ref, W_ref, bm_ref, out_ref):
  cc0 = cc_ref[:, 0:1]
  cc1 = cc_ref[:, 1:2]
  oh0 = (lax.broadcasted_iota(jnp.int32, (8, 10), 1) == cc0).astype(jnp.float32)
  oh1 = (lax.broadcasted_iota(jnp.int32, (8, 4), 1) == cc1).astype(jnp.float32)
  z = jnp.concatenate([
      x_ref[...],
      jnp.dot(oh0, e0_ref[...], preferred_element_type=jnp.float32),
      jnp.dot(oh1, e1_ref[...], preferred_element_type=jnp.float32),
  ], axis=1)
  out_ref[...] = jnp.dot(z, W_ref[...],
                         preferred_element_type=jnp.float32) + bm_ref[...]


def _stageB_body(s0_ref, mT_ref, Wr0_ref, br0_ref, Wroot0_ref, Wr1_ref,
                 Wroot1_ref, p1_ref, r1_ref, inv_ref):
  s = s0_ref[0] + s0_ref[1]
  cnt = s[:, 8:9]
  inv = 1.0 / jnp.maximum(cnt, 1.0)
  a0 = s[:, 0:8] * inv
  mt = mT_ref[...]
  for b in range(B):
    u = (a0[:, b:b + 1] * Wr0_ref[...] + mt[:, b:b + 1] * Wroot0_ref[...]
         + br0_ref[...])
    h1 = jnp.maximum(u, 0.0)
    p1_ref[:, b * 16:(b + 1) * 16] = jnp.dot(
        h1, Wr1_ref[...], preferred_element_type=jnp.float32)
    r1_ref[:, b * 16:(b + 1) * 16] = jnp.dot(
        h1, Wroot1_ref[...], preferred_element_type=jnp.float32)
  inv_ref[...] = jnp.broadcast_to(inv, (_BLK, 8))


def _stageC_body(s1_ref, r1_ref, inv_ref, br1t_ref, wr2t_ref, wroot2t_ref,
                 p2_ref, r2_ref):
  inv = inv_ref[:, 0:1]
  agg1 = (s1_ref[0] + s1_ref[1]) * inv
  h2 = jnp.maximum(agg1 + r1_ref[...] + br1t_ref[...], 0.0)
  p2cols = []
  r2cols = []
  for b in range(B):
    hb = h2[:, b * 16:(b + 1) * 16]
    p2cols.append(jnp.sum(hb * wr2t_ref[...], axis=1, keepdims=True))
    r2cols.append(jnp.sum(hb * wroot2t_ref[...], axis=1, keepdims=True))
  p2_ref[...] = jnp.concatenate(p2cols, axis=1)
  r2_ref[...] = jnp.concatenate(r2cols, axis=1)


def _stageD_body(s2_ref, inv_ref, r2_ref, br2_ref, out_ref):
  agg2 = (s2_ref[0] + s2_ref[1]) * inv_ref[:, 0:1]
  out_ref[...] = agg2 + r2_ref[...] + br2_ref[0, 0]


def _nblk(i):
  return (i, 0)


def kernel(x, categorical_covariates, edge_index, edge_weight, emb0, emb1,
           W_map, b_map, Wr0, br0, Wroot0, Wr1, br1, Wroot1, Wr2, br2,
           Wroot2):
  f32 = jnp.float32
  cc = categorical_covariates.astype(jnp.int32)
  Wp = jnp.pad(W_map, ((0, 0), (0, NT - N)))
  bm = jnp.pad(b_map, (0, NT - N)).reshape(1, NT)
  src = edge_index[0].astype(jnp.int32)
  dst = edge_index[1].astype(jnp.int32)
  ew = edge_weight.astype(f32)
  src_p = jnp.pad(src, (0, E_PAD - E)).reshape(NW, CHUNKS, K)
  dst_p = jnp.pad(dst, (0, E_PAD - E),
                  constant_values=N).reshape(NW, CHUNKS, K)
  ew_p = jnp.pad(ew, (0, E_PAD - E)).reshape(NW, CHUNKS, K)
  src_w = jnp.pad(src, (0, E_PADW - E)).reshape(NW, CHUNKSW, KW)
  dst_w = jnp.pad(dst, (0, E_PADW - E),
                  constant_values=N).reshape(NW, CHUNKSW, KW)
  ew_w = jnp.pad(ew, (0, E_PADW - E)).reshape(NW, CHUNKSW, KW)
  br1t = jnp.tile(br1, B).reshape(1, 128)
  wr2t = Wr2[:, 0].reshape(1, 16)
  wroot2t = Wroot2[:, 0].reshape(1, 16)
  br2_2d = br2.reshape(1, 1)

  full = lambda s: pl.BlockSpec(s, lambda i: tuple(0 for _ in s))

  m = pl.pallas_call(
      _stageA_body,
      grid=(_GRID,),
      in_specs=[full((8, 128)), full((8, 2)), full((10, 4)), full((4, 3)),
                pl.BlockSpec((135, _BLK), lambda i: (0, i)),
                pl.BlockSpec((1, _BLK), lambda i: (0, i))],
      out_specs=pl.BlockSpec((8, _BLK), lambda i: (0, i)),
      out_shape=jax.ShapeDtypeStruct((8, NT), f32),
  )(x, cc, emb0, emb1, Wp, bm)

  s0 = _make_spmv_narrow()(m.reshape(8 * NT), src_p, dst_p, ew_p)
  s0n = s0.reshape(2, 9, NT).transpose(0, 2, 1)

  p1, r1, invb = pl.pallas_call(
      _stageB_body,
      grid=(_GRID,),
      in_specs=[pl.BlockSpec((2, _BLK, 9), lambda i: (0, i, 0)),
                pl.BlockSpec((_BLK, 8), _nblk),
                full((1, 32)), full((1, 32)), full((1, 32)),
                full((32, 16)), full((32, 16))],
      out_specs=[pl.BlockSpec((_BLK, 128), _nblk),
                 pl.BlockSpec((_BLK, 128), _nblk),
                 pl.BlockSpec((_BLK, 8), _nblk)],
      out_shape=[jax.ShapeDtypeStruct((NT, 128), f32),
                 jax.ShapeDtypeStruct((NT, 128), f32),
                 jax.ShapeDtypeStruct((NT, 8), f32)],
  )(s0n, m.T, Wr0, br0.reshape(1, 32), Wroot0, Wr1, Wroot1)

  s1 = _make_spmv_wide()(p1, src_w, dst_w, ew_w)

  p2, r2 = pl.pallas_call(
      _stageC_body,
      grid=(_GRID,),
      in_specs=[pl.BlockSpec((2, _BLK, 128), lambda i: (0, i, 0)),
                pl.BlockSpec((_BLK, 128), _nblk),
                pl.BlockSpec((_BLK, 8), _nblk),
                full((1, 128)), full((1, 16)), full((1, 16))],
      out_specs=[pl.BlockSpec((_BLK, 8), _nblk),
                 pl.BlockSpec((_BLK, 8), _nblk)],
      out_shape=[jax.ShapeDtypeStruct((NT, 8), f32),
                 jax.ShapeDtypeStruct((NT, 8), f32)],
  )(s1, r1, invb, br1t, wr2t, wroot2t)

  s2 = _make_spmv_narrow()(p2.T.reshape(8 * NT), src_p, dst_p, ew_p)
  s2n = s2.reshape(2, 9, NT)[:, :8].transpose(0, 2, 1)

  outT = pl.pallas_call(
      _stageD_body,
      grid=(_GRID,),
      in_specs=[pl.BlockSpec((2, _BLK, 8), lambda i: (0, i, 0)),
                pl.BlockSpec((_BLK, 8), _nblk),
                pl.BlockSpec((_BLK, 8), _nblk),
                full((1, 1))],
      out_specs=pl.BlockSpec((_BLK, 8), _nblk),
      out_shape=jax.ShapeDtypeStruct((NT, 8), f32),
  )(s2n, invb, r2, br2_2d)

  return outT[:N].T
